# baseline (speedup 1.0000x reference)
import functools
import jax
import jax.numpy as jnp
from jax import lax
from jax.experimental import pallas as pl
from jax.experimental.pallas import tpu as pltpu
from jax.experimental.pallas import tpu_sc as plsc

NC = 2
NS = 16
NW = NC * NS

RING = 3


def _make_gather(B, T, D):
    assert B % NW == 0
    bpw = B // NW

    mesh = plsc.VectorSubcoreMesh(core_axis_name="c", subcore_axis_name="s")

    @functools.partial(
        pl.kernel,
        mesh=mesh,
        out_type=jax.ShapeDtypeStruct((B, T, D), jnp.float32),
        scratch_types=[
            pltpu.VMEM((bpw, T), jnp.int32),
            pltpu.VMEM((RING, T, D), jnp.float32),
            pltpu.SemaphoreType.DMA((RING,)),
            pltpu.SemaphoreType.DMA((RING,)),
        ],
        compiler_params=pltpu.CompilerParams(use_tc_tiling_on_sc=False),
    )
    def gather_kernel(idx_hbm, table_hbm, out_hbm, idx_v, rows_v, gsem, wsem):
        wid = lax.axis_index("s") * NC + lax.axis_index("c")
        b0 = wid * bpw

        pltpu.sync_copy(idx_hbm.at[pl.ds(b0, bpw)], idx_v)

        def fire_gathers(i):
            slot = lax.rem(i, RING)
            pltpu.async_copy(table_hbm.at[idx_v.at[i, pl.ds(0, 128)]],
                             rows_v.at[slot, pl.ds(0, 128)], gsem.at[slot])
            pltpu.async_copy(table_hbm.at[idx_v.at[i, pl.ds(128, T - 128)]],
                             rows_v.at[slot, pl.ds(128, T - 128)],
                             gsem.at[slot])

        def wait_gathers(slot):
            pltpu.make_async_copy(table_hbm.at[idx_v.at[0, pl.ds(0, 128)]],
                                  rows_v.at[slot, pl.ds(0, 128)],
                                  gsem.at[slot]).wait()
            pltpu.make_async_copy(
                table_hbm.at[idx_v.at[0, pl.ds(128, T - 128)]],
                rows_v.at[slot, pl.ds(128, T - 128)], gsem.at[slot]).wait()

        fire_gathers(0)
        fire_gathers(1)

        @pl.loop(0, bpw)
        def row(i):
            slot = lax.rem(i, RING)

            @pl.when(jnp.logical_and(i >= 1, i + 2 < bpw))
            def _():
                pltpu.make_async_copy(rows_v.at[lax.rem(i + 2, RING)],
                                      out_hbm.at[b0],
                                      wsem.at[lax.rem(i + 2, RING)]).wait()

            @pl.when(i + 2 < bpw)
            def _():
                fire_gathers(i + 2)

            wait_gathers(slot)
            pltpu.async_copy(rows_v.at[slot], out_hbm.at[b0 + i],
                             wsem.at[slot])

        @pl.loop(0, RING)
        def drain(k):
            pltpu.make_async_copy(rows_v.at[k], out_hbm.at[b0],
                                  wsem.at[k]).wait()

    return gather_kernel


def kernel(x, Z):
    B, T = x.shape
    V, D = Z.shape
    return _make_gather(B, T, D)(x.astype(jnp.int32), Z)


# speedup vs baseline: 1.7648x; 1.7648x over previous
import functools
import jax
import jax.numpy as jnp
from jax import lax
from jax.experimental import pallas as pl
from jax.experimental.pallas import tpu as pltpu
from jax.experimental.pallas import tpu_sc as plsc

NC = 2
NS = 16
NW = NC * NS

RING = 3
IGRP = 16
LANES = 128


def _make_gather(B, T, D):
    assert B % (NW * IGRP) == 0
    bpw = B // NW
    ngrp = bpw // IGRP

    mesh = plsc.VectorSubcoreMesh(core_axis_name="c", subcore_axis_name="s")

    @functools.partial(
        pl.kernel,
        mesh=mesh,
        out_type=jax.ShapeDtypeStruct((B * T, LANES), jnp.float32),
        scratch_types=[
            pltpu.VMEM((2, IGRP, T), jnp.int32),
            pltpu.VMEM((RING, T, D), jnp.float32),
            pltpu.SemaphoreType.DMA,
            pltpu.SemaphoreType.DMA((RING,)),
            pltpu.SemaphoreType.DMA((RING,)),
        ],
        compiler_params=pltpu.CompilerParams(use_tc_tiling_on_sc=False),
    )
    def gather_kernel(idx_hbm, table_hbm, out_hbm, idx_v, rows_v, isem, gsem,
                      wsem):
        wid = lax.axis_index("s") * NC + lax.axis_index("c")
        b0 = wid * bpw

        pltpu.sync_copy(idx_hbm.at[pl.ds(b0, IGRP)], idx_v.at[0])

        def fire_gathers(i):
            slot = lax.rem(i, RING)
            g = lax.rem(i // IGRP, 2)
            r = lax.rem(i, IGRP)
            pltpu.async_copy(table_hbm.at[idx_v.at[g, r, pl.ds(0, 128)]],
                             rows_v.at[slot, pl.ds(0, 128)],
                             gsem.at[slot])
            pltpu.async_copy(table_hbm.at[idx_v.at[g, r, pl.ds(128, T - 128)]],
                             rows_v.at[slot, pl.ds(128, T - 128)],
                             gsem.at[slot])

        def wait_gathers(slot):
            pltpu.make_async_copy(table_hbm.at[idx_v.at[0, 0, pl.ds(0, 128)]],
                                  rows_v.at[slot, pl.ds(0, 128)],
                                  gsem.at[slot]).wait()
            pltpu.make_async_copy(
                table_hbm.at[idx_v.at[0, 0, pl.ds(128, T - 128)]],
                rows_v.at[slot, pl.ds(128, T - 128)],
                gsem.at[slot]).wait()

        fire_gathers(0)
        fire_gathers(1)

        @pl.loop(0, bpw)
        def row(i):
            slot = lax.rem(i, RING)

            @pl.when(jnp.logical_and(lax.rem(i, IGRP) == 0,
                                     i + IGRP < bpw))
            def _():
                pltpu.async_copy(
                    idx_hbm.at[pl.ds(b0 + i + IGRP, IGRP)],
                    idx_v.at[lax.rem(i // IGRP + 1, 2)], isem)

            @pl.when(jnp.logical_and(lax.rem(i, IGRP) == IGRP - 2,
                                     i + IGRP < bpw + IGRP - 2))
            def _():
                pltpu.make_async_copy(idx_hbm.at[pl.ds(b0, IGRP)],
                                      idx_v.at[0], isem).wait()

            @pl.when(jnp.logical_and(i >= 1, i + 2 < bpw))
            def _():
                pltpu.make_async_copy(
                    rows_v.at[lax.rem(i + 2, RING)],
                    out_hbm.at[pl.ds(b0 * T, T), pl.ds(0, D)],
                    wsem.at[lax.rem(i + 2, RING)]).wait()

            @pl.when(i + 2 < bpw)
            def _():
                fire_gathers(i + 2)

            wait_gathers(slot)
            pltpu.async_copy(rows_v.at[slot],
                             out_hbm.at[pl.ds((b0 + i) * T, T), pl.ds(0, D)],
                             wsem.at[slot])

        @pl.loop(0, RING)
        def drain(k):
            pltpu.make_async_copy(rows_v.at[k],
                                  out_hbm.at[pl.ds(b0 * T, T), pl.ds(0, D)],
                                  wsem.at[k]).wait()

    return gather_kernel


def kernel(x, Z):
    B, T = x.shape
    V, D = Z.shape
    out = _make_gather(B, T, D)(x.astype(jnp.int32), Z)
    return out[:, :D].reshape(B, T, D)
